# baseline (speedup 1.0000x reference)
import functools

import jax
import jax.numpy as jnp
from jax import lax
from jax.experimental import pallas as pl
from jax.experimental.pallas import tpu as pltpu
from jax.experimental.pallas import tpu_sc as plsc

N = 10000
E = 320000
D = 128
L = 3
NGT = 30
NC = 2
NS = 16
E2 = 327680
ROWS2 = E2 // 128
ROWS_SC = ROWS2 // NC
NCH = N // 80


@functools.cache
def _mesh():
    return plsc.VectorSubcoreMesh(core_axis_name="c", subcore_axis_name="s",
                                  num_cores=NC, num_subcores=NS)


def _prep_run(dst2d, invdeg_out, idx8, ones_v, dv80, deg_sp, sem):
    c = lax.axis_index("c")
    s = lax.axis_index("s")

    for k in range(8):
        ones_v[pl.ds(k * 16, 16)] = jnp.full((16,), 1.0, jnp.float32)
    for k in range(5):
        dv80[pl.ds(k * 16, 16)] = jnp.zeros((16,), jnp.float32)

    @pl.when(c == 0)
    def _():
        def zbody(j, _):
            ch = s + j * NS

            @pl.when(ch < NCH)
            def _():
                pltpu.sync_copy(dv80, deg_sp.at[pl.ds(ch * 80, 80)])
            return 0
        lax.fori_loop(0, 8, zbody, 0)

        @pl.when(s == 0)
        def _():
            pltpu.sync_copy(dv80.at[pl.ds(0, 8)], deg_sp.at[pl.ds(N, 8)])

    plsc.subcore_barrier()

    @pl.when(c == 0)
    def _():
        def dbody(m, _):
            pltpu.sync_copy(dst2d.at[pl.ds(s * 160 + m * 8, 8)], idx8)
            descs = [pltpu.async_copy(ones_v, deg_sp.at[idx8.at[k]],
                                      sem, add=True)
                     for k in range(8)]
            for dsc in descs:
                dsc.wait()
            return 0
        lax.fori_loop(0, 20, dbody, 0)

    plsc.subcore_barrier()

    @pl.when(c == 0)
    def _():
        def ibody(j, _):
            ch = s + j * NS

            @pl.when(ch < NCH)
            def _():
                pltpu.sync_copy(deg_sp.at[pl.ds(ch * 80, 80)], dv80)
                for k in range(5):
                    sl = pl.ds(k * 16, 16)
                    dv80[sl] = 1.0 / jnp.maximum(dv80[sl], 1.0)
                pltpu.sync_copy(dv80, invdeg_out.at[pl.ds(ch * 80, 80)])
            return 0
        lax.fori_loop(0, 8, ibody, 0)


def _make_prep():
    @functools.partial(
        pl.kernel,
        out_type=jax.ShapeDtypeStruct((N,), jnp.float32),
        mesh=_mesh(),
        scratch_types=[
            pltpu.VMEM((8, 128), jnp.int32),
            pltpu.VMEM((128,), jnp.float32),
            pltpu.VMEM((80,), jnp.float32),
            pltpu.VMEM_SHARED((N + 8,), jnp.float32),
            pltpu.SemaphoreType.DMA,
        ],
    )
    def prep(dst2d, invdeg_out, idx8, ones_v, dv80, deg_sp, sem):
        _prep_run(dst2d, invdeg_out, idx8, ones_v, dv80, deg_sp, sem)
    return prep


def _edge_run(p_hbm, src2d, dst2d, ef3d, w1eT, z_hbm, acc_out,
              sidx, didx, p_v0, p_v1, ef_v0, ef_v1, w_v, acc_sp,
              gsem, esem, ssem):
    c = lax.axis_index("c")
    s = lax.axis_index("s")

    pltpu.sync_copy(w1eT, w_v)

    def zbody(j, _):
        ch = s + j * NS

        @pl.when(ch < NCH)
        def _():
            pltpu.sync_copy(z_hbm.at[pl.ds(ch * 80, 80)],
                            acc_sp.at[pl.ds(ch * 80, 80)])
        return 0
    lax.fori_loop(0, 8, zbody, 0)

    plsc.subcore_barrier()

    base = c * ROWS_SC + s * (ROWS_SC // NS)

    def compute(p_v, ef_v):
        def grp(g, _):
            gsl = pl.ds(g * 16, 16)
            e0g = ef_v[0, gsl]
            e1g = ef_v[1, gsl]
            e2g = ef_v[2, gsl]
            for l in range(16):
                r = g * 16 + l
                li = jnp.full((16,), l, jnp.int32)
                e0 = e0g.at[li].get(mode="promise_in_bounds")
                e1 = e1g.at[li].get(mode="promise_in_bounds")
                e2 = e2g.at[li].get(mode="promise_in_bounds")
                for kk in range(8):
                    sl = pl.ds(kk * 16, 16)
                    x = (p_v[r, sl] + e0 * w_v[0, sl]
                         + e1 * w_v[1, sl] + e2 * w_v[2, sl])
                    p_v[r, sl] = jnp.maximum(x, x * 0.01)
            return 0
        lax.fori_loop(0, 8, grp, 0)

    for blk in range(2):
        b0 = base + blk * 40
        pltpu.sync_copy(src2d.at[pl.ds(b0, 40)], sidx)
        pltpu.sync_copy(dst2d.at[pl.ds(b0, 40)], didx)

        def pair(i, _):
            r0 = 2 * i
            r1 = r0 + 1
            ga = pltpu.async_copy(p_hbm.at[sidx.at[r0]], p_v0, gsem)
            ea = pltpu.async_copy(ef3d.at[b0 + r0], ef_v0, esem)
            gb = pltpu.async_copy(p_hbm.at[sidx.at[r1]], p_v1, gsem)
            eb = pltpu.async_copy(ef3d.at[b0 + r1], ef_v1, esem)
            ga.wait()
            ea.wait()
            compute(p_v0, ef_v0)
            sa = pltpu.async_copy(p_v0, acc_sp.at[didx.at[r0]], ssem,
                                  add=True)
            gb.wait()
            eb.wait()
            compute(p_v1, ef_v1)
            sb = pltpu.async_copy(p_v1, acc_sp.at[didx.at[r1]], ssem,
                                  add=True)
            sa.wait()
            sb.wait()
            return 0
        lax.fori_loop(0, 20, pair, 0)

    plsc.subcore_barrier()

    def drain(j, _):
        ch = s + j * NS

        @pl.when(ch < NCH)
        def _():
            pltpu.sync_copy(acc_sp.at[pl.ds(ch * 80, 80)],
                            acc_out.at[c, pl.ds(ch * 80, 80)])
        return 0
    lax.fori_loop(0, 8, drain, 0)


def _make_edge():
    @functools.partial(
        pl.kernel,
        out_type=jax.ShapeDtypeStruct((NC, N, D), jnp.float32),
        mesh=_mesh(),
        scratch_types=[
            pltpu.VMEM((40, 128), jnp.int32),
            pltpu.VMEM((40, 128), jnp.int32),
            pltpu.VMEM((128, D), jnp.float32),
            pltpu.VMEM((128, D), jnp.float32),
            pltpu.VMEM((3, 128), jnp.float32),
            pltpu.VMEM((3, 128), jnp.float32),
            pltpu.VMEM((3, 128), jnp.float32),
            pltpu.VMEM_SHARED((N + 8, D), jnp.float32),
            pltpu.SemaphoreType.DMA,
            pltpu.SemaphoreType.DMA,
            pltpu.SemaphoreType.DMA,
        ],
    )
    def edge(p_hbm, src2d, dst2d, ef3d, w1eT, z_hbm, acc_out,
             sidx, didx, p_v0, p_v1, ef_v0, ef_v1, w_v, acc_sp,
             gsem, esem, ssem):
        _edge_run(p_hbm, src2d, dst2d, ef3d, w1eT, z_hbm, acc_out,
                  sidx, didx, p_v0, p_v1, ef_v0, ef_v1, w_v, acc_sp,
                  gsem, esem, ssem)
    return edge


_NB = 1000


def _p0_body(nt_ref, emb_ref, w_ref, b_ref, h0_ref, p0_ref):
    nt = nt_ref[...]
    iota = lax.broadcasted_iota(jnp.int32, (_NB, NGT), 1)
    oh = (iota == nt).astype(jnp.float32)
    h0 = jnp.dot(oh, emb_ref[...], preferred_element_type=jnp.float32)
    h0_ref[...] = h0
    p0_ref[...] = jnp.dot(h0, w_ref[...],
                          preferred_element_type=jnp.float32) + b_ref[...]


def _p0(nt, emb, wT, brow):
    return pl.pallas_call(
        _p0_body,
        grid=(N // _NB,),
        in_specs=[pl.BlockSpec((_NB, 1), lambda i: (i, 0)),
                  pl.BlockSpec((NGT, D), lambda i: (0, 0)),
                  pl.BlockSpec((D, D), lambda i: (0, 0)),
                  pl.BlockSpec((1, D), lambda i: (0, 0))],
        out_specs=[pl.BlockSpec((_NB, D), lambda i: (i, 0)),
                   pl.BlockSpec((_NB, D), lambda i: (i, 0))],
        out_shape=[jax.ShapeDtypeStruct((N, D), jnp.float32),
                   jax.ShapeDtypeStruct((N, D), jnp.float32)],
    )(nt, emb, wT, brow)


def _combine_body(h_ref, acc_ref, invd_ref, w2h_ref, w2n_ref, b2_ref,
                  w1n_ref, b1n_ref, hout_ref, pout_ref):
    hN = (acc_ref[0] + acc_ref[1]) * invd_ref[...]
    x = jnp.dot(h_ref[...], w2h_ref[...], preferred_element_type=jnp.float32)
    x = x + jnp.dot(hN, w2n_ref[...], preferred_element_type=jnp.float32)
    h2 = jnp.maximum(x + b2_ref[...], 0.0)
    hout_ref[...] = h2
    pout_ref[...] = jnp.dot(h2, w1n_ref[...],
                            preferred_element_type=jnp.float32) + b1n_ref[...]


def _combine_last_body(h_ref, acc_ref, invd_ref, w2h_ref, w2n_ref, b2_ref,
                       hout_ref):
    hN = (acc_ref[0] + acc_ref[1]) * invd_ref[...]
    x = jnp.dot(h_ref[...], w2h_ref[...], preferred_element_type=jnp.float32)
    x = x + jnp.dot(hN, w2n_ref[...], preferred_element_type=jnp.float32)
    hout_ref[...] = jnp.maximum(x + b2_ref[...], 0.0)


_W_SPECS = [pl.BlockSpec((D, D), lambda i: (0, 0)),
            pl.BlockSpec((D, D), lambda i: (0, 0)),
            pl.BlockSpec((1, D), lambda i: (0, 0))]
_IN_SPECS = [pl.BlockSpec((_NB, D), lambda i: (i, 0)),
             pl.BlockSpec((NC, _NB, D), lambda i: (0, i, 0)),
             pl.BlockSpec((_NB, 1), lambda i: (i, 0))]


def _combine(h, acc, invd, w2hT, w2nT, b2row, w1nT, b1nrow):
    return pl.pallas_call(
        _combine_body,
        grid=(N // _NB,),
        in_specs=_IN_SPECS + _W_SPECS
        + [pl.BlockSpec((D, D), lambda i: (0, 0)),
           pl.BlockSpec((1, D), lambda i: (0, 0))],
        out_specs=[pl.BlockSpec((_NB, D), lambda i: (i, 0)),
                   pl.BlockSpec((_NB, D), lambda i: (i, 0))],
        out_shape=[jax.ShapeDtypeStruct((N, D), jnp.float32),
                   jax.ShapeDtypeStruct((N, D), jnp.float32)],
    )(h, acc, invd, w2hT, w2nT, b2row, w1nT, b1nrow)


def _combine_last(h, acc, invd, w2hT, w2nT, b2row):
    return pl.pallas_call(
        _combine_last_body,
        grid=(N // _NB,),
        in_specs=_IN_SPECS + _W_SPECS,
        out_specs=pl.BlockSpec((_NB, D), lambda i: (i, 0)),
        out_shape=jax.ShapeDtypeStruct((N, D), jnp.float32),
    )(h, acc, invd, w2hT, w2nT, b2row)


def kernel(node_types, edge_index, edge_feats, embedding, W1, b1, W2, b2):
    f32 = jnp.float32
    nt = node_types.astype(jnp.int32)
    ei = edge_index.astype(jnp.int32)
    src2d = jnp.concatenate(
        [ei[0], jnp.zeros((E2 - E,), jnp.int32)]).reshape(ROWS2, 128)
    dst2d = jnp.concatenate(
        [ei[1], jnp.full((E2 - E,), N, jnp.int32)]).reshape(ROWS2, 128)
    ef_p = jnp.concatenate(
        [edge_feats.astype(f32), jnp.zeros((E2 - E, 3), f32)])
    ef3d = ef_p.reshape(ROWS2, 128, 3).transpose(0, 2, 1)
    W1hT = jnp.transpose(W1[:, :, :D], (0, 2, 1)).astype(f32)
    W1eT = jnp.transpose(W1[:, :, D:], (0, 2, 1)).astype(f32)
    W2hT = jnp.transpose(W2[:, :, :D], (0, 2, 1)).astype(f32)
    W2nT = jnp.transpose(W2[:, :, D:], (0, 2, 1)).astype(f32)
    b1r = b1.reshape(L, 1, D).astype(f32)
    b2r = b2.reshape(L, 1, D).astype(f32)
    zN = jnp.zeros((N, D), f32)

    prep = _make_prep()
    edge = _make_edge()

    invdeg = prep(dst2d)
    invd2 = invdeg.reshape(N, 1)
    h, P = _p0(nt.reshape(N, 1), embedding.astype(f32), W1hT[0], b1r[0])
    for i in range(L):
        acc = edge(P, src2d, dst2d, ef3d, W1eT[i], zN)
        if i < L - 1:
            h, P = _combine(h, acc, invd2, W2hT[i], W2nT[i], b2r[i],
                            W1hT[i + 1], b1r[i + 1])
        else:
            h = _combine_last(h, acc, invd2, W2hT[i], W2nT[i], b2r[i])
    return h


# speedup vs baseline: 1.9201x; 1.9201x over previous
import functools

import jax
import jax.numpy as jnp
from jax import lax
from jax.experimental import pallas as pl
from jax.experimental.pallas import tpu as pltpu
from jax.experimental.pallas import tpu_sc as plsc

N = 10000
E = 320000
D = 128
HD = D // 2
L = 3
NGT = 30
NC = 2
NS = 16
E2 = 327680
ROWS2 = E2 // 128
RPW = ROWS2 // NS
ROWS64 = E2 // 64
RPW64 = ROWS64 // NC // NS
NCH = N // 80


@functools.cache
def _mesh():
    return plsc.VectorSubcoreMesh(core_axis_name="c", subcore_axis_name="s",
                                  num_cores=NC, num_subcores=NS)


def _prep_run(dst2d, invdeg_out, idx8, ones_v, dv80, deg_sp, sem):
    c = lax.axis_index("c")
    s = lax.axis_index("s")

    for k in range(8):
        ones_v[pl.ds(k * 16, 16)] = jnp.full((16,), 1.0, jnp.float32)
    for k in range(5):
        dv80[pl.ds(k * 16, 16)] = jnp.zeros((16,), jnp.float32)

    @pl.when(c == 0)
    def _():
        def zbody(j, _):
            ch = s + j * NS

            @pl.when(ch < NCH)
            def _():
                pltpu.sync_copy(dv80, deg_sp.at[pl.ds(ch * 80, 80)])
            return 0
        lax.fori_loop(0, 8, zbody, 0)

        @pl.when(s == 0)
        def _():
            pltpu.sync_copy(dv80.at[pl.ds(0, 8)], deg_sp.at[pl.ds(N, 8)])

    plsc.subcore_barrier()

    @pl.when(c == 0)
    def _():
        def dbody(m, _):
            pltpu.sync_copy(dst2d.at[pl.ds(s * RPW + m * 8, 8)], idx8)
            descs = [pltpu.async_copy(ones_v, deg_sp.at[idx8.at[k]],
                                      sem, add=True)
                     for k in range(8)]
            for dsc in descs:
                dsc.wait()
            return 0
        lax.fori_loop(0, RPW // 8, dbody, 0)

    plsc.subcore_barrier()

    @pl.when(c == 0)
    def _():
        def ibody(j, _):
            ch = s + j * NS

            @pl.when(ch < NCH)
            def _():
                pltpu.sync_copy(deg_sp.at[pl.ds(ch * 80, 80)], dv80)
                for k in range(5):
                    sl = pl.ds(k * 16, 16)
                    dv80[sl] = 1.0 / jnp.maximum(dv80[sl], 1.0)
                pltpu.sync_copy(dv80, invdeg_out.at[pl.ds(ch * 80, 80)])
            return 0
        lax.fori_loop(0, 8, ibody, 0)


def _make_prep():
    @functools.partial(
        pl.kernel,
        out_type=jax.ShapeDtypeStruct((N,), jnp.float32),
        mesh=_mesh(),
        scratch_types=[
            pltpu.VMEM((8, 128), jnp.int32),
            pltpu.VMEM((128,), jnp.float32),
            pltpu.VMEM((80,), jnp.float32),
            pltpu.VMEM_SHARED((N + 8,), jnp.float32),
            pltpu.SemaphoreType.DMA,
        ],
    )
    def prep(dst2d, invdeg_out, idx8, ones_v, dv80, deg_sp, sem):
        _prep_run(dst2d, invdeg_out, idx8, ones_v, dv80, deg_sp, sem)
    return prep


def _edge_run(p_hbm, src64, dst64, c_hbm, z_hbm, acc_out,
              sidx, didx, p_v0, p_v1, c_v0, c_v1, acc_sp,
              gsem, csem, ssem):
    c = lax.axis_index("c")
    s = lax.axis_index("s")

    def zbody(j, _):
        ch = s + j * NS

        @pl.when(ch < NCH)
        def _():
            pltpu.sync_copy(z_hbm.at[pl.ds(ch * 80, 80)],
                            acc_sp.at[pl.ds(ch * 80, 80)])
        return 0
    lax.fori_loop(0, 8, zbody, 0)

    plsc.subcore_barrier()

    base = c * (ROWS64 // NC) + s * RPW64

    def compute(p_v, c_v):
        def row(r, _):
            for kk in range(8):
                sl = pl.ds(kk * 16, 16)
                x = p_v[r, sl] + c_v[r, sl]
                p_v[r, sl] = jnp.maximum(x, x * 0.01)
            return 0
        lax.fori_loop(0, 64, row, 0)

    for blk in range(4):
        b0 = base + blk * 40
        pltpu.sync_copy(src64.at[pl.ds(b0, 40)], sidx)
        pltpu.sync_copy(dst64.at[pl.ds(b0, 40)], didx)

        def pair(i, _):
            r0 = 2 * i
            r1 = r0 + 1
            ga = pltpu.async_copy(p_hbm.at[sidx.at[r0]], p_v0, gsem)
            ca = pltpu.async_copy(c_hbm.at[pl.ds((b0 + r0) * 64, 64)],
                                  c_v0, csem)
            gb = pltpu.async_copy(p_hbm.at[sidx.at[r1]], p_v1, gsem)
            cb = pltpu.async_copy(c_hbm.at[pl.ds((b0 + r1) * 64, 64)],
                                  c_v1, csem)
            ga.wait()
            ca.wait()
            compute(p_v0, c_v0)
            sa = pltpu.async_copy(p_v0, acc_sp.at[didx.at[r0]], ssem,
                                  add=True)
            gb.wait()
            cb.wait()
            compute(p_v1, c_v1)
            sb = pltpu.async_copy(p_v1, acc_sp.at[didx.at[r1]], ssem,
                                  add=True)
            sa.wait()
            sb.wait()
            return 0
        lax.fori_loop(0, 20, pair, 0)

    plsc.subcore_barrier()

    def drain(j, _):
        ch = s + j * NS

        @pl.when(ch < NCH)
        def _():
            pltpu.sync_copy(acc_sp.at[pl.ds(ch * 80, 80)],
                            acc_out.at[c, pl.ds(ch * 80, 80)])
        return 0
    lax.fori_loop(0, 8, drain, 0)


def _make_edge():
    @functools.partial(
        pl.kernel,
        out_type=jax.ShapeDtypeStruct((NC, N, D), jnp.float32),
        mesh=_mesh(),
        scratch_types=[
            pltpu.VMEM((40, 64), jnp.int32),
            pltpu.VMEM((40, 64), jnp.int32),
            pltpu.VMEM((64, D), jnp.float32),
            pltpu.VMEM((64, D), jnp.float32),
            pltpu.VMEM((64, D), jnp.float32),
            pltpu.VMEM((64, D), jnp.float32),
            pltpu.VMEM_SHARED((N + 8, D), jnp.float32),
            pltpu.SemaphoreType.DMA,
            pltpu.SemaphoreType.DMA,
            pltpu.SemaphoreType.DMA,
        ],
    )
    def edge(p_hbm, src64, dst64, c_hbm, z_hbm, acc_out,
             sidx, didx, p_v0, p_v1, c_v0, c_v1, acc_sp,
             gsem, csem, ssem):
        _edge_run(p_hbm, src64, dst64, c_hbm, z_hbm, acc_out,
                  sidx, didx, p_v0, p_v1, c_v0, c_v1, acc_sp,
                  gsem, csem, ssem)
    return edge


_EB = 2048


def _cmul_body(ef_ref, w_ref, out_ref):
    e = ef_ref[...]
    for l in range(L):
        acc = (e[:, 0:1] * w_ref[l, 0:1, :]
               + e[:, 1:2] * w_ref[l, 1:2, :]
               + e[:, 2:3] * w_ref[l, 2:3, :])
        out_ref[l] = acc


def _cmul(ef, w1eT):
    return pl.pallas_call(
        _cmul_body,
        grid=(E2 // _EB,),
        in_specs=[pl.BlockSpec((_EB, 3), lambda i: (i, 0)),
                  pl.BlockSpec((L, 3, D), lambda i: (0, 0, 0))],
        out_specs=pl.BlockSpec((L, _EB, D), lambda i: (0, i, 0)),
        out_shape=jax.ShapeDtypeStruct((L, E2, D), jnp.float32),
    )(ef, w1eT)


_NB = 1000


def _p0_body(nt_ref, emb_ref, w_ref, b_ref, h0_ref, p0_ref):
    nt = nt_ref[...]
    iota = lax.broadcasted_iota(jnp.int32, (_NB, NGT), 1)
    oh = (iota == nt).astype(jnp.float32)
    h0 = jnp.dot(oh, emb_ref[...], preferred_element_type=jnp.float32)
    h0_ref[...] = h0
    p0_ref[...] = jnp.dot(h0, w_ref[...],
                          preferred_element_type=jnp.float32) + b_ref[...]


def _p0(nt, emb, wT, brow):
    return pl.pallas_call(
        _p0_body,
        grid=(N // _NB,),
        in_specs=[pl.BlockSpec((_NB, 1), lambda i: (i, 0)),
                  pl.BlockSpec((NGT, D), lambda i: (0, 0)),
                  pl.BlockSpec((D, D), lambda i: (0, 0)),
                  pl.BlockSpec((1, D), lambda i: (0, 0))],
        out_specs=[pl.BlockSpec((_NB, D), lambda i: (i, 0)),
                   pl.BlockSpec((_NB, D), lambda i: (i, 0))],
        out_shape=[jax.ShapeDtypeStruct((N, D), jnp.float32),
                   jax.ShapeDtypeStruct((N, D), jnp.float32)],
    )(nt, emb, wT, brow)


def _combine_body(h_ref, acc_ref, invd_ref, w2h_ref, w2n_ref, b2_ref,
                  w1n_ref, b1n_ref, hout_ref, pout_ref):
    hN = (acc_ref[0] + acc_ref[1]) * invd_ref[...]
    x = jnp.dot(h_ref[...], w2h_ref[...], preferred_element_type=jnp.float32)
    x = x + jnp.dot(hN, w2n_ref[...], preferred_element_type=jnp.float32)
    h2 = jnp.maximum(x + b2_ref[...], 0.0)
    hout_ref[...] = h2
    pout_ref[...] = jnp.dot(h2, w1n_ref[...],
                            preferred_element_type=jnp.float32) + b1n_ref[...]


def _combine_last_body(h_ref, acc_ref, invd_ref, w2h_ref, w2n_ref, b2_ref,
                       hout_ref):
    hN = (acc_ref[0] + acc_ref[1]) * invd_ref[...]
    x = jnp.dot(h_ref[...], w2h_ref[...], preferred_element_type=jnp.float32)
    x = x + jnp.dot(hN, w2n_ref[...], preferred_element_type=jnp.float32)
    hout_ref[...] = jnp.maximum(x + b2_ref[...], 0.0)


_W_SPECS = [pl.BlockSpec((D, D), lambda i: (0, 0)),
            pl.BlockSpec((D, D), lambda i: (0, 0)),
            pl.BlockSpec((1, D), lambda i: (0, 0))]
_IN_SPECS = [pl.BlockSpec((_NB, D), lambda i: (i, 0)),
             pl.BlockSpec((NC, _NB, D), lambda i: (0, i, 0)),
             pl.BlockSpec((_NB, 1), lambda i: (i, 0))]


def _combine(h, acc, invd, w2hT, w2nT, b2row, w1nT, b1nrow):
    return pl.pallas_call(
        _combine_body,
        grid=(N // _NB,),
        in_specs=_IN_SPECS + _W_SPECS
        + [pl.BlockSpec((D, D), lambda i: (0, 0)),
           pl.BlockSpec((1, D), lambda i: (0, 0))],
        out_specs=[pl.BlockSpec((_NB, D), lambda i: (i, 0)),
                   pl.BlockSpec((_NB, D), lambda i: (i, 0))],
        out_shape=[jax.ShapeDtypeStruct((N, D), jnp.float32),
                   jax.ShapeDtypeStruct((N, D), jnp.float32)],
    )(h, acc, invd, w2hT, w2nT, b2row, w1nT, b1nrow)


def _combine_last(h, acc, invd, w2hT, w2nT, b2row):
    return pl.pallas_call(
        _combine_last_body,
        grid=(N // _NB,),
        in_specs=_IN_SPECS + _W_SPECS,
        out_specs=pl.BlockSpec((_NB, D), lambda i: (i, 0)),
        out_shape=jax.ShapeDtypeStruct((N, D), jnp.float32),
    )(h, acc, invd, w2hT, w2nT, b2row)


def kernel(node_types, edge_index, edge_feats, embedding, W1, b1, W2, b2):
    f32 = jnp.float32
    nt = node_types.astype(jnp.int32)
    ei = edge_index.astype(jnp.int32)
    src_p = jnp.concatenate([ei[0], jnp.zeros((E2 - E,), jnp.int32)])
    dst_p = jnp.concatenate([ei[1], jnp.full((E2 - E,), N, jnp.int32)])
    dst2d = dst_p.reshape(ROWS2, 128)
    src64 = src_p.reshape(ROWS64, 64)
    dst64 = dst_p.reshape(ROWS64, 64)
    ef_p = jnp.concatenate(
        [edge_feats.astype(f32), jnp.zeros((E2 - E, 3), f32)])
    W1hT = jnp.transpose(W1[:, :, :D], (0, 2, 1)).astype(f32)
    W1eT = jnp.transpose(W1[:, :, D:], (0, 2, 1)).astype(f32)
    W2hT = jnp.transpose(W2[:, :, :D], (0, 2, 1)).astype(f32)
    W2nT = jnp.transpose(W2[:, :, D:], (0, 2, 1)).astype(f32)
    b1r = b1.reshape(L, 1, D).astype(f32)
    b2r = b2.reshape(L, 1, D).astype(f32)
    zN = jnp.zeros((N, D), f32)

    prep = _make_prep()
    edge = _make_edge()

    invdeg = prep(dst2d)
    invd2 = invdeg.reshape(N, 1)
    C = _cmul(ef_p, W1eT)
    h, P = _p0(nt.reshape(N, 1), embedding.astype(f32), W1hT[0], b1r[0])
    for i in range(L):
        acc = edge(P, src64, dst64, C[i], zN)
        if i < L - 1:
            h, P = _combine(h, acc, invd2, W2hT[i], W2nT[i], b2r[i],
                            W1hT[i + 1], b1r[i + 1])
        else:
            h = _combine_last(h, acc, invd2, W2hT[i], W2nT[i], b2r[i])
    return h


# speedup vs baseline: 3.4854x; 1.8152x over previous
import functools

import jax
import jax.numpy as jnp
from jax import lax
from jax.experimental import pallas as pl
from jax.experimental.pallas import tpu as pltpu
from jax.experimental.pallas import tpu_sc as plsc

N = 10000
E = 320000
D = 128
HD = D // 2
L = 3
NGT = 30
NC = 2
NS = 16
E2 = 327680
ROWS2 = E2 // 128
RPW = ROWS2 // NS
ROWS64 = E2 // 64
REAL64 = E // 64
RPW64 = ROWS64 // NC // NS
NCH = N // 80


@functools.cache
def _mesh():
    return plsc.VectorSubcoreMesh(core_axis_name="c", subcore_axis_name="s",
                                  num_cores=NC, num_subcores=NS)


def _prep_run(dst2d, invdeg_out, idx8, ones_v, dv80, deg_sp, sem):
    c = lax.axis_index("c")
    s = lax.axis_index("s")

    for k in range(8):
        ones_v[pl.ds(k * 16, 16)] = jnp.full((16,), 1.0, jnp.float32)
    for k in range(5):
        dv80[pl.ds(k * 16, 16)] = jnp.zeros((16,), jnp.float32)

    @pl.when(c == 0)
    def _():
        def zbody(j, _):
            ch = s + j * NS

            @pl.when(ch < NCH)
            def _():
                pltpu.sync_copy(dv80, deg_sp.at[pl.ds(ch * 80, 80)])
            return 0
        lax.fori_loop(0, 8, zbody, 0)

        @pl.when(s == 0)
        def _():
            pltpu.sync_copy(dv80.at[pl.ds(0, 8)], deg_sp.at[pl.ds(N, 8)])

    plsc.subcore_barrier()

    @pl.when(c == 0)
    def _():
        def dbody(m, _):
            pltpu.sync_copy(dst2d.at[pl.ds(s * RPW + m * 8, 8)], idx8)
            descs = [pltpu.async_copy(ones_v, deg_sp.at[idx8.at[k]],
                                      sem, add=True)
                     for k in range(8)]
            for dsc in descs:
                dsc.wait()
            return 0
        lax.fori_loop(0, RPW // 8, dbody, 0)

    plsc.subcore_barrier()

    @pl.when(c == 0)
    def _():
        def ibody(j, _):
            ch = s + j * NS

            @pl.when(ch < NCH)
            def _():
                pltpu.sync_copy(deg_sp.at[pl.ds(ch * 80, 80)], dv80)
                for k in range(5):
                    sl = pl.ds(k * 16, 16)
                    dv80[sl] = 1.0 / jnp.maximum(dv80[sl], 1.0)
                pltpu.sync_copy(dv80, invdeg_out.at[pl.ds(ch * 80, 80)])
            return 0
        lax.fori_loop(0, 8, ibody, 0)


def _make_prep():
    @functools.partial(
        pl.kernel,
        out_type=jax.ShapeDtypeStruct((N,), jnp.float32),
        mesh=_mesh(),
        scratch_types=[
            pltpu.VMEM((8, 128), jnp.int32),
            pltpu.VMEM((128,), jnp.float32),
            pltpu.VMEM((80,), jnp.float32),
            pltpu.VMEM_SHARED((N + 8,), jnp.float32),
            pltpu.SemaphoreType.DMA,
        ],
    )
    def prep(dst2d, invdeg_out, idx8, ones_v, dv80, deg_sp, sem):
        _prep_run(dst2d, invdeg_out, idx8, ones_v, dv80, deg_sp, sem)
    return prep


def _edge_run(p_hbm, src64, dst64, c_hbm, z_hbm, acc_out,
              sidx, didx, p_v0, p_v1, c_v0, c_v1, acc_sp,
              gsem, csem, ssem):
    c = lax.axis_index("c")
    s = lax.axis_index("s")

    def zbody(j, _):
        ch = s + j * NS

        @pl.when(ch < NCH)
        def _():
            pltpu.sync_copy(z_hbm.at[pl.ds(ch * 80, 80)],
                            acc_sp.at[pl.ds(ch * 80, 80)])
        return 0
    lax.fori_loop(0, 8, zbody, 0)

    plsc.subcore_barrier()

    base = c * (ROWS64 // NC) + s * RPW64

    def compute(p_v, c_v):
        def row(r, _):
            for kk in range(8):
                sl = pl.ds(kk * 16, 16)
                x = p_v[r, sl] + c_v[r, sl]
                p_v[r, sl] = jnp.maximum(x, x * 0.01)
            return 0
        lax.fori_loop(0, 64, row, 0)

    for blk in range(4):
        b0 = base + blk * 40

        @pl.when(b0 < REAL64)
        def _(b0=b0):
            pltpu.sync_copy(src64.at[pl.ds(b0, 40)], sidx)
            pltpu.sync_copy(dst64.at[pl.ds(b0, 40)], didx)

            def pair(i, _):
                r0 = 2 * i
                r1 = r0 + 1
                ga = pltpu.async_copy(p_hbm.at[sidx.at[r0]], p_v0, gsem)
                ca = pltpu.async_copy(c_hbm.at[pl.ds((b0 + r0) * 64, 64)],
                                      c_v0, csem)
                gb = pltpu.async_copy(p_hbm.at[sidx.at[r1]], p_v1, gsem)
                cb = pltpu.async_copy(c_hbm.at[pl.ds((b0 + r1) * 64, 64)],
                                      c_v1, csem)
                ga.wait()
                ca.wait()
                compute(p_v0, c_v0)
                sa = pltpu.async_copy(p_v0, acc_sp.at[didx.at[r0]], ssem,
                                      add=True)
                gb.wait()
                cb.wait()
                compute(p_v1, c_v1)
                sb = pltpu.async_copy(p_v1, acc_sp.at[didx.at[r1]], ssem,
                                      add=True)
                sa.wait()
                sb.wait()
                return 0
            lax.fori_loop(0, 20, pair, 0)

    plsc.subcore_barrier()

    def drain(j, _):
        ch = s + j * NS

        @pl.when(ch < NCH)
        def _():
            pltpu.sync_copy(acc_sp.at[pl.ds(ch * 80, 80)],
                            acc_out.at[c, pl.ds(ch * 80, 80)])
        return 0
    lax.fori_loop(0, 8, drain, 0)


def _make_edge():
    @functools.partial(
        pl.kernel,
        out_type=jax.ShapeDtypeStruct((NC, N, D), jnp.float32),
        mesh=_mesh(),
        scratch_types=[
            pltpu.VMEM((40, 64), jnp.int32),
            pltpu.VMEM((40, 64), jnp.int32),
            pltpu.VMEM((64, D), jnp.float32),
            pltpu.VMEM((64, D), jnp.float32),
            pltpu.VMEM((64, D), jnp.float32),
            pltpu.VMEM((64, D), jnp.float32),
            pltpu.VMEM_SHARED((N + 8, D), jnp.float32),
            pltpu.SemaphoreType.DMA,
            pltpu.SemaphoreType.DMA,
            pltpu.SemaphoreType.DMA,
        ],
    )
    def edge(p_hbm, src64, dst64, c_hbm, z_hbm, acc_out,
             sidx, didx, p_v0, p_v1, c_v0, c_v1, acc_sp,
             gsem, csem, ssem):
        _edge_run(p_hbm, src64, dst64, c_hbm, z_hbm, acc_out,
                  sidx, didx, p_v0, p_v1, c_v0, c_v1, acc_sp,
                  gsem, csem, ssem)
    return edge


_EB = 2048


def _cmul_body(ef_ref, w_ref, out_ref):
    e = ef_ref[...]
    out_ref[...] = (e[:, 0:1] * w_ref[0:1, :]
                    + e[:, 1:2] * w_ref[1:2, :]
                    + e[:, 2:3] * w_ref[2:3, :])


def _cmul(ef, w1eT_l):
    return pl.pallas_call(
        _cmul_body,
        grid=(E2 // _EB,),
        in_specs=[pl.BlockSpec((_EB, 3), lambda i: (i, 0)),
                  pl.BlockSpec((3, D), lambda i: (0, 0))],
        out_specs=pl.BlockSpec((_EB, D), lambda i: (i, 0)),
        out_shape=jax.ShapeDtypeStruct((E2, D), jnp.float32),
    )(ef, w1eT_l)


_NB = 1000


def _p0_body(nt_ref, emb_ref, w_ref, b_ref, h0_ref, p0_ref):
    nt = nt_ref[...]
    iota = lax.broadcasted_iota(jnp.int32, (_NB, NGT), 1)
    oh = (iota == nt).astype(jnp.float32)
    h0 = jnp.dot(oh, emb_ref[...], preferred_element_type=jnp.float32)
    h0_ref[...] = h0
    p0_ref[...] = jnp.dot(h0, w_ref[...],
                          preferred_element_type=jnp.float32) + b_ref[...]


def _p0(nt, emb, wT, brow):
    return pl.pallas_call(
        _p0_body,
        grid=(N // _NB,),
        in_specs=[pl.BlockSpec((_NB, 1), lambda i: (i, 0)),
                  pl.BlockSpec((NGT, D), lambda i: (0, 0)),
                  pl.BlockSpec((D, D), lambda i: (0, 0)),
                  pl.BlockSpec((1, D), lambda i: (0, 0))],
        out_specs=[pl.BlockSpec((_NB, D), lambda i: (i, 0)),
                   pl.BlockSpec((_NB, D), lambda i: (i, 0))],
        out_shape=[jax.ShapeDtypeStruct((N, D), jnp.float32),
                   jax.ShapeDtypeStruct((N, D), jnp.float32)],
    )(nt, emb, wT, brow)


def _combine_body(h_ref, acc_ref, invd_ref, w2h_ref, w2n_ref, b2_ref,
                  w1n_ref, b1n_ref, hout_ref, pout_ref):
    hN = (acc_ref[0] + acc_ref[1]) * invd_ref[...]
    x = jnp.dot(h_ref[...], w2h_ref[...], preferred_element_type=jnp.float32)
    x = x + jnp.dot(hN, w2n_ref[...], preferred_element_type=jnp.float32)
    h2 = jnp.maximum(x + b2_ref[...], 0.0)
    hout_ref[...] = h2
    pout_ref[...] = jnp.dot(h2, w1n_ref[...],
                            preferred_element_type=jnp.float32) + b1n_ref[...]


def _combine_last_body(h_ref, acc_ref, invd_ref, w2h_ref, w2n_ref, b2_ref,
                       hout_ref):
    hN = (acc_ref[0] + acc_ref[1]) * invd_ref[...]
    x = jnp.dot(h_ref[...], w2h_ref[...], preferred_element_type=jnp.float32)
    x = x + jnp.dot(hN, w2n_ref[...], preferred_element_type=jnp.float32)
    hout_ref[...] = jnp.maximum(x + b2_ref[...], 0.0)


_W_SPECS = [pl.BlockSpec((D, D), lambda i: (0, 0)),
            pl.BlockSpec((D, D), lambda i: (0, 0)),
            pl.BlockSpec((1, D), lambda i: (0, 0))]
_IN_SPECS = [pl.BlockSpec((_NB, D), lambda i: (i, 0)),
             pl.BlockSpec((NC, _NB, D), lambda i: (0, i, 0)),
             pl.BlockSpec((_NB, 1), lambda i: (i, 0))]


def _combine(h, acc, invd, w2hT, w2nT, b2row, w1nT, b1nrow):
    return pl.pallas_call(
        _combine_body,
        grid=(N // _NB,),
        in_specs=_IN_SPECS + _W_SPECS
        + [pl.BlockSpec((D, D), lambda i: (0, 0)),
           pl.BlockSpec((1, D), lambda i: (0, 0))],
        out_specs=[pl.BlockSpec((_NB, D), lambda i: (i, 0)),
                   pl.BlockSpec((_NB, D), lambda i: (i, 0))],
        out_shape=[jax.ShapeDtypeStruct((N, D), jnp.float32),
                   jax.ShapeDtypeStruct((N, D), jnp.float32)],
    )(h, acc, invd, w2hT, w2nT, b2row, w1nT, b1nrow)


def _combine_last(h, acc, invd, w2hT, w2nT, b2row):
    return pl.pallas_call(
        _combine_last_body,
        grid=(N // _NB,),
        in_specs=_IN_SPECS + _W_SPECS,
        out_specs=pl.BlockSpec((_NB, D), lambda i: (i, 0)),
        out_shape=jax.ShapeDtypeStruct((N, D), jnp.float32),
    )(h, acc, invd, w2hT, w2nT, b2row)


def kernel(node_types, edge_index, edge_feats, embedding, W1, b1, W2, b2):
    f32 = jnp.float32
    nt = node_types.astype(jnp.int32)
    ei = edge_index.astype(jnp.int32)
    src_p = jnp.concatenate([ei[0], jnp.zeros((E2 - E,), jnp.int32)])
    dst_p = jnp.concatenate([ei[1], jnp.full((E2 - E,), N, jnp.int32)])
    dst2d = dst_p.reshape(ROWS2, 128)
    src64 = src_p.reshape(ROWS64, 64)
    dst64 = dst_p.reshape(ROWS64, 64)
    ef_p = jnp.concatenate(
        [edge_feats.astype(f32), jnp.zeros((E2 - E, 3), f32)])
    W1hT = jnp.transpose(W1[:, :, :D], (0, 2, 1)).astype(f32)
    W1eT = jnp.transpose(W1[:, :, D:], (0, 2, 1)).astype(f32)
    W2hT = jnp.transpose(W2[:, :, :D], (0, 2, 1)).astype(f32)
    W2nT = jnp.transpose(W2[:, :, D:], (0, 2, 1)).astype(f32)
    b1r = b1.reshape(L, 1, D).astype(f32)
    b2r = b2.reshape(L, 1, D).astype(f32)
    zN = jnp.zeros((N, D), f32)

    prep = _make_prep()
    edge = _make_edge()

    invdeg = prep(dst2d)
    invd2 = invdeg.reshape(N, 1)
    Cs = [_cmul(ef_p, W1eT[i]) for i in range(L)]
    h, P = _p0(nt.reshape(N, 1), embedding.astype(f32), W1hT[0], b1r[0])
    for i in range(L):
        acc = edge(P, src64, dst64, Cs[i], zN)
        if i < L - 1:
            h, P = _combine(h, acc, invd2, W2hT[i], W2nT[i], b2r[i],
                            W1hT[i + 1], b1r[i + 1])
        else:
            h = _combine_last(h, acc, invd2, W2hT[i], W2nT[i], b2r[i])
    return h


# speedup vs baseline: 3.7922x; 1.0880x over previous
import functools

import jax
import jax.numpy as jnp
from jax import lax
from jax.experimental import pallas as pl
from jax.experimental.pallas import tpu as pltpu
from jax.experimental.pallas import tpu_sc as plsc

N = 10000
E = 320000
D = 128
HD = D // 2
L = 3
NGT = 30
NC = 2
NS = 16
REAL64 = E // 64
PBLK = 625
RPW64 = 160
NCH = N // 80


@functools.cache
def _mesh():
    return plsc.VectorSubcoreMesh(core_axis_name="c", subcore_axis_name="s",
                                  num_cores=NC, num_subcores=NS)


def _prep_run(dst5k, invdeg_out, idx8, ones_v, dv80, deg_sp, sem):
    c = lax.axis_index("c")
    s = lax.axis_index("s")

    for k in range(4):
        ones_v[pl.ds(k * 16, 16)] = jnp.full((16,), 1.0, jnp.float32)
    for k in range(5):
        dv80[pl.ds(k * 16, 16)] = jnp.zeros((16,), jnp.float32)

    @pl.when(c == 0)
    def _():
        def zbody(j, _):
            ch = s + j * NS

            @pl.when(ch < NCH)
            def _():
                pltpu.sync_copy(dv80, deg_sp.at[pl.ds(ch * 80, 80)])
            return 0
        lax.fori_loop(0, 8, zbody, 0)

        @pl.when(s == 0)
        def _():
            pltpu.sync_copy(dv80.at[pl.ds(0, 8)], deg_sp.at[pl.ds(N, 8)])

    plsc.subcore_barrier()

    @pl.when(c == 0)
    def _():
        def dbody(j, _):
            b = s + j * NS

            @pl.when(b < PBLK)
            def _():
                pltpu.sync_copy(dst5k.at[pl.ds(b * 8, 8)], idx8)
                descs = [pltpu.async_copy(ones_v, deg_sp.at[idx8.at[k]],
                                          sem, add=True)
                         for k in range(8)]
                for dsc in descs:
                    dsc.wait()
            return 0
        lax.fori_loop(0, (PBLK + NS - 1) // NS, dbody, 0)

    plsc.subcore_barrier()

    @pl.when(c == 0)
    def _():
        def ibody(j, _):
            ch = s + j * NS

            @pl.when(ch < NCH)
            def _():
                pltpu.sync_copy(deg_sp.at[pl.ds(ch * 80, 80)], dv80)
                for k in range(5):
                    sl = pl.ds(k * 16, 16)
                    dv80[sl] = 1.0 / jnp.maximum(dv80[sl], 1.0)
                pltpu.sync_copy(dv80, invdeg_out.at[pl.ds(ch * 80, 80)])
            return 0
        lax.fori_loop(0, 8, ibody, 0)


def _make_prep():
    @functools.partial(
        pl.kernel,
        out_type=jax.ShapeDtypeStruct((N,), jnp.float32),
        mesh=_mesh(),
        scratch_types=[
            pltpu.VMEM((8, 64), jnp.int32),
            pltpu.VMEM((64,), jnp.float32),
            pltpu.VMEM((80,), jnp.float32),
            pltpu.VMEM_SHARED((N + 8,), jnp.float32),
            pltpu.SemaphoreType.DMA,
        ],
    )
    def prep(dst5k, invdeg_out, idx8, ones_v, dv80, deg_sp, sem):
        _prep_run(dst5k, invdeg_out, idx8, ones_v, dv80, deg_sp, sem)
    return prep


def _edge_run(p_hbm, src64, dst64, c_hbm, z_hbm, acc_out,
              sidx, didx, p_v0, p_v1, c_v0, c_v1, acc_sp,
              gsem, csem, ssem):
    c = lax.axis_index("c")
    s = lax.axis_index("s")

    def zbody(j, _):
        ch = s + j * NS

        @pl.when(ch < NCH)
        def _():
            pltpu.sync_copy(z_hbm.at[pl.ds(ch * 80, 80)],
                            acc_sp.at[pl.ds(ch * 80, 80)])
        return 0
    lax.fori_loop(0, 8, zbody, 0)

    plsc.subcore_barrier()

    base = c * (NS * RPW64) + s * RPW64

    def compute(p_v, c_v):
        def row(r, _):
            for kk in range(8):
                sl = pl.ds(kk * 16, 16)
                x = p_v[r, sl] + c_v[r, sl]
                p_v[r, sl] = jnp.maximum(x, x * 0.01)
            return 0
        lax.fori_loop(0, 64, row, 0)

    for blk in range(4):
        b0 = base + blk * 40

        @pl.when(b0 < REAL64)
        def _(b0=b0):
            pltpu.sync_copy(src64.at[pl.ds(b0, 40)], sidx)
            pltpu.sync_copy(dst64.at[pl.ds(b0, 40)], didx)

            def pair(i, _):
                r0 = 2 * i
                r1 = r0 + 1

                @pl.when(i > 0)
                def _():
                    pltpu.make_async_copy(z_hbm.at[pl.ds(0, 64)], p_v0,
                                          ssem).wait()
                    pltpu.make_async_copy(z_hbm.at[pl.ds(0, 64)], p_v1,
                                          ssem).wait()
                ga = pltpu.async_copy(p_hbm.at[sidx.at[r0]], p_v0, gsem)
                ca = pltpu.async_copy(c_hbm.at[pl.ds((b0 + r0) * 64, 64)],
                                      c_v0, csem)
                gb = pltpu.async_copy(p_hbm.at[sidx.at[r1]], p_v1, gsem)
                cb = pltpu.async_copy(c_hbm.at[pl.ds((b0 + r1) * 64, 64)],
                                      c_v1, csem)
                ga.wait()
                ca.wait()
                compute(p_v0, c_v0)
                pltpu.async_copy(p_v0, acc_sp.at[didx.at[r0]], ssem,
                                 add=True)
                gb.wait()
                cb.wait()
                compute(p_v1, c_v1)
                pltpu.async_copy(p_v1, acc_sp.at[didx.at[r1]], ssem,
                                 add=True)
                return 0
            lax.fori_loop(0, 20, pair, 0)
            pltpu.make_async_copy(z_hbm.at[pl.ds(0, 64)], p_v0, ssem).wait()
            pltpu.make_async_copy(z_hbm.at[pl.ds(0, 64)], p_v1, ssem).wait()

    plsc.subcore_barrier()

    def drain(j, _):
        ch = s + j * NS

        @pl.when(ch < NCH)
        def _():
            pltpu.sync_copy(acc_sp.at[pl.ds(ch * 80, 80)],
                            acc_out.at[c, pl.ds(ch * 80, 80)])
        return 0
    lax.fori_loop(0, 8, drain, 0)


def _make_edge():
    @functools.partial(
        pl.kernel,
        out_type=jax.ShapeDtypeStruct((NC, N, D), jnp.float32),
        mesh=_mesh(),
        scratch_types=[
            pltpu.VMEM((40, 64), jnp.int32),
            pltpu.VMEM((40, 64), jnp.int32),
            pltpu.VMEM((64, D), jnp.float32),
            pltpu.VMEM((64, D), jnp.float32),
            pltpu.VMEM((64, D), jnp.float32),
            pltpu.VMEM((64, D), jnp.float32),
            pltpu.VMEM_SHARED((N, D), jnp.float32),
            pltpu.SemaphoreType.DMA,
            pltpu.SemaphoreType.DMA,
            pltpu.SemaphoreType.DMA,
        ],
    )
    def edge(p_hbm, src64, dst64, c_hbm, z_hbm, acc_out,
             sidx, didx, p_v0, p_v1, c_v0, c_v1, acc_sp,
             gsem, csem, ssem):
        _edge_run(p_hbm, src64, dst64, c_hbm, z_hbm, acc_out,
                  sidx, didx, p_v0, p_v1, c_v0, c_v1, acc_sp,
                  gsem, csem, ssem)
    return edge


_EB = 2000


def _cmul_body(ef_ref, w_ref, out_ref):
    e = ef_ref[...]
    out_ref[...] = (e[:, 0:1] * w_ref[0:1, :]
                    + e[:, 1:2] * w_ref[1:2, :]
                    + e[:, 2:3] * w_ref[2:3, :])


def _cmul(ef, w1eT_l):
    return pl.pallas_call(
        _cmul_body,
        grid=(E // _EB,),
        in_specs=[pl.BlockSpec((_EB, 3), lambda i: (i, 0)),
                  pl.BlockSpec((3, D), lambda i: (0, 0))],
        out_specs=pl.BlockSpec((_EB, D), lambda i: (i, 0)),
        out_shape=jax.ShapeDtypeStruct((E, D), jnp.float32),
    )(ef, w1eT_l)


_NB = 1000


def _p0_body(nt_ref, emb_ref, w_ref, b_ref, h0_ref, p0_ref):
    nt = nt_ref[...]
    iota = lax.broadcasted_iota(jnp.int32, (_NB, NGT), 1)
    oh = (iota == nt).astype(jnp.float32)
    h0 = jnp.dot(oh, emb_ref[...], preferred_element_type=jnp.float32)
    h0_ref[...] = h0
    p0_ref[...] = jnp.dot(h0, w_ref[...],
                          preferred_element_type=jnp.float32) + b_ref[...]


def _p0(nt, emb, wT, brow):
    return pl.pallas_call(
        _p0_body,
        grid=(N // _NB,),
        in_specs=[pl.BlockSpec((_NB, 1), lambda i: (i, 0)),
                  pl.BlockSpec((NGT, D), lambda i: (0, 0)),
                  pl.BlockSpec((D, D), lambda i: (0, 0)),
                  pl.BlockSpec((1, D), lambda i: (0, 0))],
        out_specs=[pl.BlockSpec((_NB, D), lambda i: (i, 0)),
                   pl.BlockSpec((_NB, D), lambda i: (i, 0))],
        out_shape=[jax.ShapeDtypeStruct((N, D), jnp.float32),
                   jax.ShapeDtypeStruct((N, D), jnp.float32)],
    )(nt, emb, wT, brow)


def _combine_body(h_ref, acc_ref, invd_ref, w2h_ref, w2n_ref, b2_ref,
                  w1n_ref, b1n_ref, hout_ref, pout_ref):
    hN = (acc_ref[0] + acc_ref[1]) * invd_ref[...]
    x = jnp.dot(h_ref[...], w2h_ref[...], preferred_element_type=jnp.float32)
    x = x + jnp.dot(hN, w2n_ref[...], preferred_element_type=jnp.float32)
    h2 = jnp.maximum(x + b2_ref[...], 0.0)
    hout_ref[...] = h2
    pout_ref[...] = jnp.dot(h2, w1n_ref[...],
                            preferred_element_type=jnp.float32) + b1n_ref[...]


def _combine_last_body(h_ref, acc_ref, invd_ref, w2h_ref, w2n_ref, b2_ref,
                       hout_ref):
    hN = (acc_ref[0] + acc_ref[1]) * invd_ref[...]
    x = jnp.dot(h_ref[...], w2h_ref[...], preferred_element_type=jnp.float32)
    x = x + jnp.dot(hN, w2n_ref[...], preferred_element_type=jnp.float32)
    hout_ref[...] = jnp.maximum(x + b2_ref[...], 0.0)


_W_SPECS = [pl.BlockSpec((D, D), lambda i: (0, 0)),
            pl.BlockSpec((D, D), lambda i: (0, 0)),
            pl.BlockSpec((1, D), lambda i: (0, 0))]
_IN_SPECS = [pl.BlockSpec((_NB, D), lambda i: (i, 0)),
             pl.BlockSpec((NC, _NB, D), lambda i: (0, i, 0)),
             pl.BlockSpec((_NB, 1), lambda i: (i, 0))]


def _combine(h, acc, invd, w2hT, w2nT, b2row, w1nT, b1nrow):
    return pl.pallas_call(
        _combine_body,
        grid=(N // _NB,),
        in_specs=_IN_SPECS + _W_SPECS
        + [pl.BlockSpec((D, D), lambda i: (0, 0)),
           pl.BlockSpec((1, D), lambda i: (0, 0))],
        out_specs=[pl.BlockSpec((_NB, D), lambda i: (i, 0)),
                   pl.BlockSpec((_NB, D), lambda i: (i, 0))],
        out_shape=[jax.ShapeDtypeStruct((N, D), jnp.float32),
                   jax.ShapeDtypeStruct((N, D), jnp.float32)],
    )(h, acc, invd, w2hT, w2nT, b2row, w1nT, b1nrow)


def _combine_last(h, acc, invd, w2hT, w2nT, b2row):
    return pl.pallas_call(
        _combine_last_body,
        grid=(N // _NB,),
        in_specs=_IN_SPECS + _W_SPECS,
        out_specs=pl.BlockSpec((_NB, D), lambda i: (i, 0)),
        out_shape=jax.ShapeDtypeStruct((N, D), jnp.float32),
    )(h, acc, invd, w2hT, w2nT, b2row)


def kernel(node_types, edge_index, edge_feats, embedding, W1, b1, W2, b2):
    f32 = jnp.float32
    nt = node_types.astype(jnp.int32)
    ei = edge_index.astype(jnp.int32)
    src64 = ei[0].reshape(REAL64, 64)
    dst64 = ei[1].reshape(REAL64, 64)
    ef_p = edge_feats.astype(f32)
    W1hT = jnp.transpose(W1[:, :, :D], (0, 2, 1)).astype(f32)
    W1eT = jnp.transpose(W1[:, :, D:], (0, 2, 1)).astype(f32)
    W2hT = jnp.transpose(W2[:, :, :D], (0, 2, 1)).astype(f32)
    W2nT = jnp.transpose(W2[:, :, D:], (0, 2, 1)).astype(f32)
    b1r = b1.reshape(L, 1, D).astype(f32)
    b2r = b2.reshape(L, 1, D).astype(f32)
    zN = jnp.zeros((N, D), f32)

    prep = _make_prep()
    edge = _make_edge()

    invdeg = prep(dst64)
    invd2 = invdeg.reshape(N, 1)
    Cs = [_cmul(ef_p, W1eT[i]) for i in range(L)]
    h, P = _p0(nt.reshape(N, 1), embedding.astype(f32), W1hT[0], b1r[0])
    for i in range(L):
        acc = edge(P, src64, dst64, Cs[i], zN)
        if i < L - 1:
            h, P = _combine(h, acc, invd2, W2hT[i], W2nT[i], b2r[i],
                            W1hT[i + 1], b1r[i + 1])
        else:
            h = _combine_last(h, acc, invd2, W2hT[i], W2nT[i], b2r[i])
    return h
